# baseline (speedup 1.0000x reference)
import functools
import math

import jax
import jax.numpy as jnp
from jax import lax
from jax.experimental import pallas as pl
from jax.experimental.pallas import tpu as pltpu
from jax.experimental.pallas import tpu_sc as plsc

VOCAB = 100000
D_MODEL = 1024
SCALE = math.sqrt(D_MODEL)
LANES = 16
VECS_PER_ROW = D_MODEL // LANES

NUM_CORES = 2
NUM_SUBCORES = 16
NW = NUM_CORES * NUM_SUBCORES

B_TOTAL = 16384
B_PER_W = B_TOTAL // NW
CHUNK = 32
NCHUNK = B_PER_W // CHUNK
NBUF = 3
GSPLIT = 2


def _emb_body(idx_hbm, table_hbm, out_hbm, idx_v, rows_v, gsems, osems):
    wid = lax.axis_index("s") * NUM_CORES + lax.axis_index("c")
    base = wid * B_PER_W
    w_per_row = 4096 // B_PER_W
    pltpu.sync_copy(
        idx_hbm.at[wid // w_per_row, pl.ds((wid % w_per_row) * B_PER_W, B_PER_W)],
        idx_v,
    )

    GPART = CHUNK // GSPLIT

    def start_gather(n, b):
        for g in range(GSPLIT):
            pltpu.async_copy(
                table_hbm.at[idx_v.at[pl.ds(n * CHUNK + g * GPART, GPART)]],
                rows_v.at[b].at[pl.ds(g * GPART, GPART)],
                gsems.at[b],
            )

    def wait_gather(b):
        pltpu.make_async_copy(
            table_hbm.at[idx_v.at[pl.ds(0, CHUNK)]], rows_v.at[b], gsems.at[b]
        ).wait()

    def start_out(c, b):
        pltpu.async_copy(
            rows_v.at[b], out_hbm.at[pl.ds(base + c * CHUNK, CHUNK)], osems.at[b]
        )

    def wait_out(b):
        pltpu.make_async_copy(
            rows_v.at[b], out_hbm.at[pl.ds(0, CHUNK)], osems.at[b]
        ).wait()

    for n in range(NBUF - 1):
        start_gather(n, n)

    def chunk_step(c, _):
        b = c % NBUF
        bn = (c + NBUF - 1) % NBUF

        @pl.when(c + NBUF - 1 < NCHUNK)
        def _():
            @pl.when(c >= 1)
            def _():
                wait_out(bn)

            start_gather(c + NBUF - 1, bn)

        wait_gather(b)

        @plsc.parallel_loop(0, CHUNK, step=1)
        def _(r):
            for v in range(VECS_PER_ROW):
                sl = pl.ds(v * LANES, LANES)
                rows_v[b, r, sl] = rows_v[b, r, sl] * SCALE

        start_out(c, b)
        return 0

    lax.fori_loop(0, NCHUNK, chunk_step, 0)

    for n in range(NCHUNK - NBUF, NCHUNK):
        wait_out(n % NBUF)


@jax.jit
def _embed(x, table):
    mesh = plsc.VectorSubcoreMesh(core_axis_name="c", subcore_axis_name="s")
    out = pl.kernel(
        _emb_body,
        out_type=jax.ShapeDtypeStruct((B_TOTAL, D_MODEL), jnp.float32),
        mesh=mesh,
        scratch_types=[
            pltpu.VMEM((B_PER_W,), jnp.int32),
            pltpu.VMEM((NBUF, CHUNK, D_MODEL), jnp.float32),
            pltpu.SemaphoreType.DMA((NBUF,)),
            pltpu.SemaphoreType.DMA((NBUF,)),
        ],
    )(x, table)
    return out


def kernel(x, table):
    out = _embed(x, table)
    return out.reshape(x.shape[0], x.shape[1], D_MODEL)


# speedup vs baseline: 1.0479x; 1.0479x over previous
import functools
import math

import jax
import jax.numpy as jnp
from jax import lax
from jax.experimental import pallas as pl
from jax.experimental.pallas import tpu as pltpu
from jax.experimental.pallas import tpu_sc as plsc

VOCAB = 100000
D_MODEL = 1024
SCALE = math.sqrt(D_MODEL)
LANES = 16
VECS_PER_ROW = D_MODEL // LANES

NUM_CORES = 2
NUM_SUBCORES = 16
NW = NUM_CORES * NUM_SUBCORES

B_TOTAL = 16384
B_PER_W = B_TOTAL // NW
CHUNK = 16
NCHUNK = B_PER_W // CHUNK
NBUF = 7
QCHUNK = 16


def _emb_body(idx_hbm, table_hbm, out_hbm, idx_v, rows_v, gsems, osems):
    wid = lax.axis_index("s") * NUM_CORES + lax.axis_index("c")
    base = wid * B_PER_W
    w_per_row = 4096 // B_PER_W
    pltpu.sync_copy(
        idx_hbm.at[wid // w_per_row, pl.ds((wid % w_per_row) * B_PER_W, B_PER_W)],
        idx_v,
    )

    def start_gather(n, b):
        pltpu.async_copy(
            table_hbm.at[idx_v.at[pl.ds(n * CHUNK, CHUNK)]],
            rows_v.at[b],
            gsems.at[b],
        )

    def wait_gather(b):
        pltpu.make_async_copy(
            table_hbm.at[idx_v.at[pl.ds(0, CHUNK)]], rows_v.at[b], gsems.at[b]
        ).wait()

    def start_out(c, b):
        pltpu.async_copy(
            rows_v.at[b], out_hbm.at[pl.ds(base + c * CHUNK, CHUNK)], osems.at[b]
        )

    def wait_out(b):
        pltpu.make_async_copy(
            rows_v.at[b], out_hbm.at[pl.ds(0, CHUNK)], osems.at[b]
        ).wait()

    for n in range(NBUF - 1):
        start_gather(n, n)

    def chunk_step(c, _):
        b = c % NBUF
        bn = (c + NBUF - 1) % NBUF

        @pl.when(c + NBUF - 1 < NCHUNK)
        def _():
            @pl.when(c >= 1)
            def _():
                wait_out(bn)

            start_gather(c + NBUF - 1, bn)

        wait_gather(b)

        @plsc.parallel_loop(0, CHUNK, step=1)
        def _(r):
            for v in range(VECS_PER_ROW):
                sl = pl.ds(v * LANES, LANES)
                rows_v[b, r, sl] = rows_v[b, r, sl] * SCALE

        start_out(c, b)
        return 0

    lax.fori_loop(0, NCHUNK, chunk_step, 0)

    for n in range(NCHUNK - NBUF, NCHUNK):
        wait_out(n % NBUF)


@jax.jit
def _embed(x, table):
    mesh = plsc.VectorSubcoreMesh(core_axis_name="c", subcore_axis_name="s")
    out = pl.kernel(
        _emb_body,
        out_type=jax.ShapeDtypeStruct((B_TOTAL, D_MODEL), jnp.float32),
        mesh=mesh,
        scratch_types=[
            pltpu.VMEM((B_PER_W,), jnp.int32),
            pltpu.VMEM((NBUF, CHUNK, D_MODEL), jnp.float32),
            pltpu.SemaphoreType.DMA((NBUF,)),
            pltpu.SemaphoreType.DMA((NBUF,)),
        ],
    )(x, table)
    return out


def kernel(x, table):
    out = _embed(x, table)
    return out.reshape(x.shape[0], x.shape[1], D_MODEL)
